# baseline (speedup 1.0000x reference)
import jax
import jax.numpy as jnp
import numpy as np
from jax.experimental import pallas as pl
from jax.experimental.pallas import tpu as pltpu

B, STOCK, T, FEAT = 2, 2048, 8, 128
NHID, RNN, NHEAD, DK, DV = 64, 64, 4, 16, 16
NEDGE = 256
NCLASS = 3
NBT = B * T
NROW = B * STOCK
TM = 256

F32 = jnp.float32
BF16 = jnp.bfloat16


def _dot(a, b):
    return jnp.dot(a, b, preferred_element_type=F32)


def _bdot(a, b):
    return jnp.dot(a.astype(jnp.bfloat16), b.astype(jnp.bfloat16),
                   preferred_element_type=F32)


def _hnew_body(h_ref, w_ref, b_ref, o_ref):
    o_ref[...] = _dot(h_ref[...], w_ref[...]) + b_ref[...]


def _hnew(H, m_W, m_b2):
    return pl.pallas_call(
        _hnew_body,
        grid=(STOCK // TM,),
        in_specs=[
            pl.BlockSpec((TM, NEDGE), lambda i: (i, 0)),
            pl.BlockSpec((NEDGE, STOCK), lambda i: (0, 0)),
            pl.BlockSpec((1, STOCK), lambda i: (0, 0)),
        ],
        out_specs=pl.BlockSpec((TM, STOCK), lambda i: (i, 0)),
        out_shape=jax.ShapeDtypeStruct((STOCK, STOCK), F32),
    )(H, m_W, m_b2)


def _xw_body(x_ref, w_ref, o_ref):
    x = x_ref[...]
    w = w_ref[...]
    o_ref[...] = jnp.concatenate(
        [_dot(x[:, :FEAT], w), _dot(x[:, FEAT:], w)], axis=1)


def _xw(X_r, gc1_W):
    return pl.pallas_call(
        _xw_body,
        grid=(STOCK // TM, NBT // 2),
        in_specs=[
            pl.BlockSpec((TM, 2 * FEAT), lambda i, j: (i, j)),
            pl.BlockSpec((FEAT, NHID), lambda i, j: (0, 0)),
        ],
        out_specs=pl.BlockSpec((TM, 2 * NHID), lambda i, j: (i, j)),
        out_shape=jax.ShapeDtypeStruct((STOCK, NBT * NHID), F32),
    )(X_r, gc1_W)


def _u_body(adj_ref, hn_ref, xw_ref, b_ref, h1_ref, h2_ref):
    xw = xw_ref[...]
    b = b_ref[...]
    h1_ref[...] = jax.nn.relu(_dot(adj_ref[...], xw) + b)
    h2_ref[...] = jax.nn.relu(_dot(hn_ref[...], xw) + b)


def _u_stage(adj, H_new, XW_r, b1c):
    KW = NBT * NHID
    return pl.pallas_call(
        _u_body,
        grid=(STOCK // TM,),
        in_specs=[
            pl.BlockSpec((TM, STOCK), lambda i: (i, 0)),
            pl.BlockSpec((TM, STOCK), lambda i: (i, 0)),
            pl.BlockSpec((STOCK, KW), lambda i: (0, 0)),
            pl.BlockSpec((1, KW), lambda i: (0, 0)),
        ],
        out_specs=[
            pl.BlockSpec((TM, KW), lambda i: (i, 0)),
            pl.BlockSpec((TM, KW), lambda i: (i, 0)),
        ],
        out_shape=[
            jax.ShapeDtypeStruct((STOCK, KW), F32),
            jax.ShapeDtypeStruct((STOCK, KW), F32),
        ],
    )(adj, H_new, XW_r, b1c)


def _v_body(adj_ref, hn_ref, h1_ref, h2_ref, v_ref):
    v_ref[...] = _dot(adj_ref[...], h1_ref[...]) + _dot(hn_ref[...], h2_ref[...])


def _v_stage(adj, H_new, Ha, Hh):
    KW = NBT * NHID
    return pl.pallas_call(
        _v_body,
        grid=(STOCK // TM,),
        in_specs=[
            pl.BlockSpec((TM, STOCK), lambda i: (i, 0)),
            pl.BlockSpec((TM, STOCK), lambda i: (i, 0)),
            pl.BlockSpec((STOCK, KW), lambda i: (0, 0)),
            pl.BlockSpec((STOCK, KW), lambda i: (0, 0)),
        ],
        out_specs=pl.BlockSpec((TM, KW), lambda i: (i, 0)),
        out_shape=jax.ShapeDtypeStruct((STOCK, KW), F32),
    )(adj, H_new, Ha, Hh)


def _gru_body(v_ref, wf0_ref, bf0_ref, whh0_ref, bhh0_ref,
              wih1_ref, bih1_ref, whh1_ref, bhh1_ref, out_ref):
    V = v_ref[...]
    wf0 = wf0_ref[...]
    bf0 = bf0_ref[...]
    whh0 = whh0_ref[...]
    bhh0 = bhh0_ref[...]
    wih1 = wih1_ref[...]
    bih1 = bih1_ref[...]
    whh1 = whh1_ref[...]
    bhh1 = bhh1_ref[...]

    def cell(gi, h, whh, bhh):
        gh = _dot(h, whh) + bhh
        r = jax.nn.sigmoid(gi[:, :RNN] + gh[:, :RNN])
        u = jax.nn.sigmoid(gi[:, RNN:2 * RNN] + gh[:, RNN:2 * RNN])
        c = jnp.tanh(gi[:, 2 * RNN:] + r * gh[:, 2 * RNN:])
        return (1.0 - u) * c + u * h

    h = jnp.zeros((TM, RNN), F32)
    outs0 = []
    for t in range(T):
        gi = _dot(V[:, t * RNN:(t + 1) * RNN], wf0) + bf0
        h = cell(gi, h, whh0, bhh0)
        outs0.append(h)
    h = jnp.zeros((TM, RNN), F32)
    for t in range(T):
        gi = _dot(outs0[t], wih1) + bih1
        h = cell(gi, h, whh1, bhh1)
        out_ref[:, t * RNN:(t + 1) * RNN] = h


def _gru(V, Wf0, bf0, Whh0T, bhh0, Wih1T, bih1, Whh1T, bhh1):
    TW = T * RNN
    wspec = pl.BlockSpec((RNN, 3 * RNN), lambda i, b: (0, 0))
    bspec = pl.BlockSpec((1, 3 * RNN), lambda i, b: (0, 0))
    return pl.pallas_call(
        _gru_body,
        grid=(STOCK // TM, B),
        in_specs=[
            pl.BlockSpec((TM, TW), lambda i, b: (i, b)),
            wspec, bspec, wspec, bspec, wspec, bspec, wspec, bspec,
        ],
        out_specs=pl.BlockSpec((TM, TW), lambda i, b: (b * (STOCK // TM) + i, 0)),
        out_shape=jax.ShapeDtypeStruct((NROW, TW), F32),
    )(V, Wf0, bf0, Whh0T, bhh0, Wih1T, bih1, Whh1T, bhh1)


TMA = 256


def _attn_body(x_ref, wq_ref, wk_ref, wv_ref, fca_ref, o_ref):
    X = x_ref[...]
    wq = wq_ref[...]
    wk = wk_ref[...]
    wv = wv_ref[...]
    fca = fca_ref[...]

    qs, ks, vs = [], [], []
    for t in range(T):
        rt = X[:, t * RNN:(t + 1) * RNN]
        qs.append(_dot(rt, wq))
        ks.append(_dot(rt, wk))
        vs.append(_dot(rt, wv))

    scale = 1.0 / np.sqrt(DK)

    def pair_score(i, j):
        prod = qs[i] * ks[j]
        return jnp.sum(prod.reshape(TMA, NHEAD, DK), axis=2) * scale

    o_cols = []
    for i in range(T):
        srow = [pair_score(i, j) for j in range(i + 1)]
        m = srow[0]
        for sj in srow[1:]:
            m = jnp.maximum(m, sj)
        es = [jnp.exp(sj - m) for sj in srow]
        tot = es[0]
        for ej in es[1:]:
            tot = tot + ej
        inv = 1.0 / tot
        o_i = jnp.zeros((TMA, NHEAD * DV), F32)
        for j in range(i + 1):
            a = (es[j] * inv)
            a64 = jnp.repeat(a[:, :, None], DV, axis=2).reshape(TMA, NHEAD * DV)
            o_i = o_i + a64 * vs[j]
        o_cols.append(_dot(o_i, fca))
    o_ref[...] = jnp.concatenate(o_cols, axis=1)


def _attn(rnn_out, wq, wk, wv, fc_attn):
    TW = T * RNN
    wspec = pl.BlockSpec((RNN, RNN), lambda i: (0, 0))
    return pl.pallas_call(
        _attn_body,
        grid=(NROW // TMA,),
        in_specs=[
            pl.BlockSpec((TMA, TW), lambda i: (i, 0)),
            wspec, wspec, wspec, wspec,
        ],
        out_specs=pl.BlockSpec((TMA, TW), lambda i: (i, 0)),
        out_shape=jax.ShapeDtypeStruct((NROW, TW), F32),
    )(rnn_out, wq, wk, wv, fc_attn)


def _mhgn_body(h_ref, ht_ref, adj_ref, o_ref):
    o_ref[...] = adj_ref[...] + _dot(h_ref[...], ht_ref[...]) * (1.0 / NEDGE)


def _mhgn(H, H_T, adj):
    return pl.pallas_call(
        _mhgn_body,
        grid=(STOCK // TM,),
        in_specs=[
            pl.BlockSpec((TM, NEDGE), lambda i: (i, 0)),
            pl.BlockSpec((NEDGE, STOCK), lambda i: (0, 0)),
            pl.BlockSpec((TM, STOCK), lambda i: (i, 0)),
        ],
        out_specs=pl.BlockSpec((TM, STOCK), lambda i: (i, 0)),
        out_shape=jax.ShapeDtypeStruct((STOCK, STOCK), F32),
    )(H, H_T, adj)


def _uenc_body(oc0_ref, oc1_ref, r0_ref, r1_ref, lng_ref, lnb_ref,
               w_ref, o_ref):
    w = w_ref[...]
    g = lng_ref[...]
    bb = lnb_ref[...]

    def enc_block(oc, rn):
        cols = []
        for t in range(T):
            et = oc[:, t * RNN:(t + 1) * RNN] + rn[:, t * RNN:(t + 1) * RNN]
            mu = jnp.mean(et, axis=-1, keepdims=True)
            var = jnp.mean((et - mu) ** 2, axis=-1, keepdims=True)
            cols.append(g * (et - mu) / jnp.sqrt(var + 1e-6) + bb)
        return _dot(jnp.concatenate(cols, axis=1), w)

    o_ref[...] = jnp.concatenate(
        [enc_block(oc0_ref[...], r0_ref[...]),
         enc_block(oc1_ref[...], r1_ref[...])], axis=1)


def _uenc(oc, rnn_out, ln_g2, ln_b2, Wh1):
    TW = T * RNN
    nb = STOCK // TM
    lo = pl.BlockSpec((TM, TW), lambda i: (i, 0))
    hi = pl.BlockSpec((TM, TW), lambda i: (nb + i, 0))
    vspec = pl.BlockSpec((1, RNN), lambda i: (0, 0))
    return pl.pallas_call(
        _uenc_body,
        grid=(nb,),
        in_specs=[
            lo, hi, lo, hi, vspec, vspec,
            pl.BlockSpec((TW, NHID), lambda i: (0, 0)),
        ],
        out_specs=pl.BlockSpec((TM, B * NHID), lambda i: (i, 0)),
        out_shape=jax.ShapeDtypeStruct((STOCK, B * NHID), F32),
    )(oc, oc, rnn_out, rnn_out, ln_g2, ln_b2, Wh1)


def _hgn_body(m_ref, u_ref, b_ref, o_ref):
    o_ref[...] = jax.nn.relu(_dot(m_ref[...], u_ref[...]) + b_ref[...])


def _hgn(M, U, bh1c):
    KW = B * NHID
    return pl.pallas_call(
        _hgn_body,
        grid=(STOCK // TM,),
        in_specs=[
            pl.BlockSpec((TM, STOCK), lambda i: (i, 0)),
            pl.BlockSpec((STOCK, KW), lambda i: (0, 0)),
            pl.BlockSpec((1, KW), lambda i: (0, 0)),
        ],
        out_specs=pl.BlockSpec((TM, KW), lambda i: (i, 0)),
        out_shape=jax.ShapeDtypeStruct((STOCK, KW), F32),
    )(M, U, bh1c)


def _final_body(hg_ref, avw_ref, avb_ref, avu_ref, lw_ref, lb_ref,
                fcwh_ref, fcwa_ref, fcb_ref, wprj_ref, seq_ref, pred_ref):
    Hg = hg_ref[...]
    a_laten = jnp.tanh(_dot(Hg, avw_ref[...]) + avb_ref[...])
    s = jnp.sum(a_laten * avu_ref[...], axis=1, keepdims=True)
    m = jnp.max(s)
    e = jnp.exp(s - m)
    alph = e / jnp.sum(e)
    acs = alph * jnp.sum(Hg, axis=1, keepdims=True)
    a_con = acs * lw_ref[...] + lb_ref[...]
    pred_ref[...] = _dot(Hg, fcwh_ref[...]) + _dot(a_con, fcwa_ref[...]) + fcb_ref[...]
    seq_ref[...] = _dot(Hg, wprj_ref[...]) * (RNN ** -0.5)


def _final(hgn_flat, av_w, av_b2, av_u2, L_W, L_b2, fcW_h, fcW_a, fc_b2, W_prjT):
    return pl.pallas_call(
        _final_body,
        out_shape=[
            jax.ShapeDtypeStruct((NROW, NCLASS), F32),
            jax.ShapeDtypeStruct((NROW, NCLASS), F32),
        ],
    )(hgn_flat, av_w, av_b2, av_u2, L_W, L_b2, fcW_h, fcW_a, fc_b2, W_prjT)


def kernel(src_seq, H, adj, n_hid, gc1_W, gc1_b, gc2_W, gc2_b, m_W, m_b,
           lin_W, lin_b, gru_Wih0, gru_Whh0, gru_bih0, gru_bhh0,
           gru_Wih1, gru_Whh1, gru_bih1, gru_bhh1, wq, wk, wv, fc_attn,
           ln_g, ln_b, Wh1, bh1, av_w, av_b, av_u, L_W, L_b, fc_W, fc_b,
           W_prj):
    del n_hid

    W2L = gc2_W @ lin_W
    bias2L = 2.0 * (gc2_b @ lin_W) + lin_b
    Wf0 = W2L @ gru_Wih0.T
    bf0 = (bias2L @ gru_Wih0.T + gru_bih0)[None, :]
    Whh0T = gru_Whh0.T
    Wih1T = gru_Wih1.T
    Whh1T = gru_Whh1.T
    bhh0 = gru_bhh0[None, :]
    bih1 = gru_bih1[None, :]
    bhh1 = gru_bhh1[None, :]

    X_r = jnp.transpose(src_seq, (1, 0, 2, 3)).reshape(STOCK, NBT * FEAT)
    b1c = jnp.tile(gc1_b, (NBT,))[None, :]
    H_new = _hnew(H, m_W, m_b[None, :])
    XW_r = _xw(X_r, gc1_W)
    Ha, Hh = _u_stage(adj, H_new, XW_r, b1c)
    V = _v_stage(adj, H_new, Ha, Hh)

    rnn_out = _gru(V, Wf0, bf0, Whh0T, bhh0, Wih1T, bih1, Whh1T, bhh1)
    oc = _attn(rnn_out, wq, wk, wv, fc_attn)

    M = _mhgn(H, H.T, adj)
    U = _uenc(oc, rnn_out, ln_g[None, :], ln_b[None, :], Wh1)
    hgn_cols = _hgn(M, U, jnp.tile(bh1, (B,))[None, :])
    hgn_flat = jnp.transpose(
        hgn_cols.reshape(STOCK, B, NHID), (1, 0, 2)).reshape(NROW, NHID)

    seq_logit, pred = _final(
        hgn_flat, av_w, av_b[None, :], av_u[None, :], L_W, L_b[None, :],
        fc_W[:NHID, :], fc_W[NHID:, :], fc_b[None, :], W_prj.T)
    return (seq_logit, pred)


# speedup vs baseline: 3.2038x; 3.2038x over previous
import jax
import jax.numpy as jnp
import numpy as np
from jax.experimental import pallas as pl
from jax.experimental.pallas import tpu as pltpu

B, STOCK, T, FEAT = 2, 2048, 8, 128
NHID, RNN, NHEAD, DK, DV = 64, 64, 4, 16, 16
NEDGE = 256
NCLASS = 3
NBT = B * T
NROW = B * STOCK
TM = 256

F32 = jnp.float32
BF16 = jnp.bfloat16


def _dot(a, b):
    return jnp.dot(a, b, preferred_element_type=F32)


def _bdot(a, b):
    return jnp.dot(a.astype(jnp.bfloat16), b.astype(jnp.bfloat16),
                   preferred_element_type=F32)


def _hnew_body(h_ref, w_ref, b_ref, o_ref):
    o_ref[...] = _dot(h_ref[...], w_ref[...]) + b_ref[...]


def _hnew(H, m_W, m_b2):
    return pl.pallas_call(
        _hnew_body,
        grid=(STOCK // TM,),
        in_specs=[
            pl.BlockSpec((TM, NEDGE), lambda i: (i, 0)),
            pl.BlockSpec((NEDGE, STOCK), lambda i: (0, 0)),
            pl.BlockSpec((1, STOCK), lambda i: (0, 0)),
        ],
        out_specs=pl.BlockSpec((TM, STOCK), lambda i: (i, 0)),
        out_shape=jax.ShapeDtypeStruct((STOCK, STOCK), F32),
    )(H, m_W, m_b2)


def _xw_body(x_ref, w_ref, o_ref):
    x = x_ref[...]
    w = w_ref[...]
    o_ref[...] = jnp.concatenate(
        [_dot(x[:, :FEAT], w), _dot(x[:, FEAT:], w)], axis=1)


def _xw(X_r, gc1_W):
    return pl.pallas_call(
        _xw_body,
        grid=(STOCK // TM, NBT // 2),
        in_specs=[
            pl.BlockSpec((TM, 2 * FEAT), lambda i, j: (i, j)),
            pl.BlockSpec((FEAT, NHID), lambda i, j: (0, 0)),
        ],
        out_specs=pl.BlockSpec((TM, 2 * NHID), lambda i, j: (i, j)),
        out_shape=jax.ShapeDtypeStruct((STOCK, NBT * NHID), F32),
    )(X_r, gc1_W)


def _u_body(adj_ref, hn_ref, xw_ref, b_ref, h1_ref, h2_ref):
    xw = xw_ref[...]
    b = b_ref[...]
    h1_ref[...] = jax.nn.relu(_dot(adj_ref[...], xw) + b)
    h2_ref[...] = jax.nn.relu(_dot(hn_ref[...], xw) + b)


def _u_stage(adj, H_new, XW_r, b1c):
    KW = NBT * NHID
    return pl.pallas_call(
        _u_body,
        grid=(STOCK // TM,),
        in_specs=[
            pl.BlockSpec((TM, STOCK), lambda i: (i, 0)),
            pl.BlockSpec((TM, STOCK), lambda i: (i, 0)),
            pl.BlockSpec((STOCK, KW), lambda i: (0, 0)),
            pl.BlockSpec((1, KW), lambda i: (0, 0)),
        ],
        out_specs=[
            pl.BlockSpec((TM, KW), lambda i: (i, 0)),
            pl.BlockSpec((TM, KW), lambda i: (i, 0)),
        ],
        out_shape=[
            jax.ShapeDtypeStruct((STOCK, KW), F32),
            jax.ShapeDtypeStruct((STOCK, KW), F32),
        ],
    )(adj, H_new, XW_r, b1c)


def _v_body(adj_ref, hn_ref, h1_ref, h2_ref, v_ref):
    v_ref[...] = _dot(adj_ref[...], h1_ref[...]) + _dot(hn_ref[...], h2_ref[...])


def _v_stage(adj, H_new, Ha, Hh):
    KW = NBT * NHID
    return pl.pallas_call(
        _v_body,
        grid=(STOCK // TM,),
        in_specs=[
            pl.BlockSpec((TM, STOCK), lambda i: (i, 0)),
            pl.BlockSpec((TM, STOCK), lambda i: (i, 0)),
            pl.BlockSpec((STOCK, KW), lambda i: (0, 0)),
            pl.BlockSpec((STOCK, KW), lambda i: (0, 0)),
        ],
        out_specs=pl.BlockSpec((TM, KW), lambda i: (i, 0)),
        out_shape=jax.ShapeDtypeStruct((STOCK, KW), F32),
    )(adj, H_new, Ha, Hh)


TMG = 512


def _gru_body(v_ref, wf0_ref, bf0_ref, whh0_ref, bhh0_ref,
              wih1_ref, bih1_ref, whh1_ref, bhh1_ref, out_ref):
    V = v_ref[...]
    wf0 = wf0_ref[...]
    bf0 = bf0_ref[...]
    whh0 = whh0_ref[...]
    bhh0 = bhh0_ref[...]
    wih1 = wih1_ref[...]
    bih1 = bih1_ref[...]
    whh1 = whh1_ref[...]
    bhh1 = bhh1_ref[...]

    def cell(gi, h, whh, bhh):
        gh = _dot(h, whh) + bhh
        r = jax.nn.sigmoid(gi[:, :RNN] + gh[:, :RNN])
        u = jax.nn.sigmoid(gi[:, RNN:2 * RNN] + gh[:, RNN:2 * RNN])
        c = jnp.tanh(gi[:, 2 * RNN:] + r * gh[:, 2 * RNN:])
        return (1.0 - u) * c + u * h

    h = jnp.zeros((TMG, RNN), F32)
    outs0 = []
    for t in range(T):
        gi = _dot(V[:, t * RNN:(t + 1) * RNN], wf0) + bf0
        h = cell(gi, h, whh0, bhh0)
        outs0.append(h)
    h = jnp.zeros((TMG, RNN), F32)
    for t in range(T):
        gi = _dot(outs0[t], wih1) + bih1
        h = cell(gi, h, whh1, bhh1)
        out_ref[:, t * RNN:(t + 1) * RNN] = h


def _gru(V, Wf0, bf0, Whh0T, bhh0, Wih1T, bih1, Whh1T, bhh1):
    TW = T * RNN
    wspec = pl.BlockSpec((RNN, 3 * RNN), lambda i, b: (0, 0))
    bspec = pl.BlockSpec((1, 3 * RNN), lambda i, b: (0, 0))
    return pl.pallas_call(
        _gru_body,
        grid=(STOCK // TMG, B),
        in_specs=[
            pl.BlockSpec((TMG, TW), lambda i, b: (i, b)),
            wspec, bspec, wspec, bspec, wspec, bspec, wspec, bspec,
        ],
        out_specs=pl.BlockSpec((TMG, TW),
                               lambda i, b: (b * (STOCK // TMG) + i, 0)),
        out_shape=jax.ShapeDtypeStruct((NROW, TW), F32),
    )(V, Wf0, bf0, Whh0T, bhh0, Wih1T, bih1, Whh1T, bhh1)


TMA = 256


def _attn_body(x_ref, wq_ref, wk_ref, wv_ref, fca_ref, hb_ref, o_ref):
    X = x_ref[...]
    wq = wq_ref[...]
    wk = wk_ref[...]
    wv = wv_ref[...]
    fca = fca_ref[...]
    HB = hb_ref[...]

    qs, ks, vs = [], [], []
    for t in range(T):
        rt = X[:, t * RNN:(t + 1) * RNN]
        qs.append(_dot(rt, wq))
        ks.append(_dot(rt, wk))
        vs.append(_dot(rt, wv))

    o_cols = []
    for i in range(T):
        srow = [_dot(qs[i] * ks[j], HB) for j in range(i + 1)]
        m = srow[0]
        for sj in srow[1:]:
            m = jnp.maximum(m, sj)
        es = [jnp.exp(sj - m) for sj in srow]
        tot = es[0]
        for ej in es[1:]:
            tot = tot + ej
        inv = 1.0 / tot
        o_i = es[0] * inv * vs[0]
        for j in range(1, i + 1):
            o_i = o_i + es[j] * inv * vs[j]
        o_cols.append(_dot(o_i, fca))
    o_ref[...] = jnp.concatenate(o_cols, axis=1)


def _attn(rnn_out, wq, wk, wv, fc_attn, HB):
    TW = T * RNN
    wspec = pl.BlockSpec((RNN, RNN), lambda i: (0, 0))
    return pl.pallas_call(
        _attn_body,
        grid=(NROW // TMA,),
        in_specs=[
            pl.BlockSpec((TMA, TW), lambda i: (i, 0)),
            wspec, wspec, wspec, wspec, wspec,
        ],
        out_specs=pl.BlockSpec((TMA, TW), lambda i: (i, 0)),
        out_shape=jax.ShapeDtypeStruct((NROW, TW), F32),
    )(rnn_out, wq, wk, wv, fc_attn, HB)


def _mhgn_body(h_ref, ht_ref, adj_ref, o_ref):
    o_ref[...] = adj_ref[...] + _dot(h_ref[...], ht_ref[...]) * (1.0 / NEDGE)


def _mhgn(H, H_T, adj):
    return pl.pallas_call(
        _mhgn_body,
        grid=(STOCK // TM,),
        in_specs=[
            pl.BlockSpec((TM, NEDGE), lambda i: (i, 0)),
            pl.BlockSpec((NEDGE, STOCK), lambda i: (0, 0)),
            pl.BlockSpec((TM, STOCK), lambda i: (i, 0)),
        ],
        out_specs=pl.BlockSpec((TM, STOCK), lambda i: (i, 0)),
        out_shape=jax.ShapeDtypeStruct((STOCK, STOCK), F32),
    )(H, H_T, adj)


def _uenc_body(oc0_ref, oc1_ref, r0_ref, r1_ref, lng_ref, lnb_ref,
               w_ref, o_ref):
    w = w_ref[...]
    g = lng_ref[...]
    bb = lnb_ref[...]

    def enc_block(oc, rn):
        cols = []
        for t in range(T):
            et = oc[:, t * RNN:(t + 1) * RNN] + rn[:, t * RNN:(t + 1) * RNN]
            mu = jnp.mean(et, axis=-1, keepdims=True)
            var = jnp.mean((et - mu) ** 2, axis=-1, keepdims=True)
            cols.append(g * (et - mu) / jnp.sqrt(var + 1e-6) + bb)
        return _dot(jnp.concatenate(cols, axis=1), w)

    o_ref[...] = jnp.concatenate(
        [enc_block(oc0_ref[...], r0_ref[...]),
         enc_block(oc1_ref[...], r1_ref[...])], axis=1)


def _uenc(oc, rnn_out, ln_g2, ln_b2, Wh1):
    TW = T * RNN
    nb = STOCK // TM
    lo = pl.BlockSpec((TM, TW), lambda i: (i, 0))
    hi = pl.BlockSpec((TM, TW), lambda i: (nb + i, 0))
    vspec = pl.BlockSpec((1, RNN), lambda i: (0, 0))
    return pl.pallas_call(
        _uenc_body,
        grid=(nb,),
        in_specs=[
            lo, hi, lo, hi, vspec, vspec,
            pl.BlockSpec((TW, NHID), lambda i: (0, 0)),
        ],
        out_specs=pl.BlockSpec((TM, B * NHID), lambda i: (i, 0)),
        out_shape=jax.ShapeDtypeStruct((STOCK, B * NHID), F32),
    )(oc, oc, rnn_out, rnn_out, ln_g2, ln_b2, Wh1)


def _hgn_body(m_ref, u_ref, b_ref, o_ref):
    o_ref[...] = jax.nn.relu(_dot(m_ref[...], u_ref[...]) + b_ref[...])


def _hgn(M, U, bh1c):
    KW = B * NHID
    return pl.pallas_call(
        _hgn_body,
        grid=(STOCK // TM,),
        in_specs=[
            pl.BlockSpec((TM, STOCK), lambda i: (i, 0)),
            pl.BlockSpec((STOCK, KW), lambda i: (0, 0)),
            pl.BlockSpec((1, KW), lambda i: (0, 0)),
        ],
        out_specs=pl.BlockSpec((TM, KW), lambda i: (i, 0)),
        out_shape=jax.ShapeDtypeStruct((STOCK, KW), F32),
    )(M, U, bh1c)


def _final_body(hg_ref, avw_ref, avb_ref, avu_ref, lw_ref, lb_ref,
                fcwh_ref, fcwa_ref, fcb_ref, wprj_ref, seq_ref, pred_ref):
    Hg = hg_ref[...]
    a_laten = jnp.tanh(_dot(Hg, avw_ref[...]) + avb_ref[...])
    s = jnp.sum(a_laten * avu_ref[...], axis=1, keepdims=True)
    m = jnp.max(s)
    e = jnp.exp(s - m)
    alph = e / jnp.sum(e)
    acs = alph * jnp.sum(Hg, axis=1, keepdims=True)
    a_con = acs * lw_ref[...] + lb_ref[...]
    pred_ref[...] = _dot(Hg, fcwh_ref[...]) + _dot(a_con, fcwa_ref[...]) + fcb_ref[...]
    seq_ref[...] = _dot(Hg, wprj_ref[...]) * (RNN ** -0.5)


def _final(hgn_flat, av_w, av_b2, av_u2, L_W, L_b2, fcW_h, fcW_a, fc_b2, W_prjT):
    return pl.pallas_call(
        _final_body,
        out_shape=[
            jax.ShapeDtypeStruct((NROW, NCLASS), F32),
            jax.ShapeDtypeStruct((NROW, NCLASS), F32),
        ],
    )(hgn_flat, av_w, av_b2, av_u2, L_W, L_b2, fcW_h, fcW_a, fc_b2, W_prjT)


def kernel(src_seq, H, adj, n_hid, gc1_W, gc1_b, gc2_W, gc2_b, m_W, m_b,
           lin_W, lin_b, gru_Wih0, gru_Whh0, gru_bih0, gru_bhh0,
           gru_Wih1, gru_Whh1, gru_bih1, gru_bhh1, wq, wk, wv, fc_attn,
           ln_g, ln_b, Wh1, bh1, av_w, av_b, av_u, L_W, L_b, fc_W, fc_b,
           W_prj):
    del n_hid

    W2L = gc2_W @ lin_W
    bias2L = 2.0 * (gc2_b @ lin_W) + lin_b
    Wf0 = W2L @ gru_Wih0.T
    bf0 = (bias2L @ gru_Wih0.T + gru_bih0)[None, :]
    Whh0T = gru_Whh0.T
    Wih1T = gru_Wih1.T
    Whh1T = gru_Whh1.T
    bhh0 = gru_bhh0[None, :]
    bih1 = gru_bih1[None, :]
    bhh1 = gru_bhh1[None, :]

    X_r = jnp.transpose(src_seq, (1, 0, 2, 3)).reshape(STOCK, NBT * FEAT)
    b1c = jnp.tile(gc1_b, (NBT,))[None, :]
    H_new = _hnew(H, m_W, m_b[None, :])
    XW_r = _xw(X_r, gc1_W)
    Ha, Hh = _u_stage(adj, H_new, XW_r, b1c)
    V = _v_stage(adj, H_new, Ha, Hh)

    rnn_out = _gru(V, Wf0, bf0, Whh0T, bhh0, Wih1T, bih1, Whh1T, bhh1)
    hb = jnp.repeat(jnp.repeat(jnp.eye(NHEAD, dtype=F32), DK, axis=0),
                    DK, axis=1) * (1.0 / np.sqrt(DK))
    oc = _attn(rnn_out, wq, wk, wv, fc_attn, hb)

    M = _mhgn(H, H.T, adj)
    U = _uenc(oc, rnn_out, ln_g[None, :], ln_b[None, :], Wh1)
    hgn_cols = _hgn(M, U, jnp.tile(bh1, (B,))[None, :])
    hgn_flat = jnp.transpose(
        hgn_cols.reshape(STOCK, B, NHID), (1, 0, 2)).reshape(NROW, NHID)

    seq_logit, pred = _final(
        hgn_flat, av_w, av_b[None, :], av_u[None, :], L_W, L_b[None, :],
        fc_W[:NHID, :], fc_W[NHID:, :], fc_b[None, :], W_prj.T)
    return (seq_logit, pred)


# speedup vs baseline: 3.2352x; 1.0098x over previous
import jax
import jax.numpy as jnp
import numpy as np
from jax.experimental import pallas as pl
from jax.experimental.pallas import tpu as pltpu

B, STOCK, T, FEAT = 2, 2048, 8, 128
NHID, RNN, NHEAD, DK, DV = 64, 64, 4, 16, 16
NEDGE = 256
NCLASS = 3
NBT = B * T
NROW = B * STOCK
TM = 256

F32 = jnp.float32
BF16 = jnp.bfloat16


def _dot(a, b):
    return jnp.dot(a, b, preferred_element_type=F32)


def _hnew_body(h_ref, w_ref, b_ref, o_ref):
    o_ref[...] = (_dot(h_ref[...], w_ref[...]) + b_ref[...]).astype(BF16)


def _hnew(H, m_W, m_b2):
    return pl.pallas_call(
        _hnew_body,
        grid=(STOCK // TM,),
        in_specs=[
            pl.BlockSpec((TM, NEDGE), lambda i: (i, 0)),
            pl.BlockSpec((NEDGE, STOCK), lambda i: (0, 0)),
            pl.BlockSpec((1, STOCK), lambda i: (0, 0)),
        ],
        out_specs=pl.BlockSpec((TM, STOCK), lambda i: (i, 0)),
        out_shape=jax.ShapeDtypeStruct((STOCK, STOCK), BF16),
    )(H, m_W, m_b2)


def _xw_body(x_ref, w_ref, o_ref):
    x = x_ref[...]
    w = w_ref[...]
    o_ref[...] = jnp.concatenate(
        [_dot(x[:, :FEAT], w), _dot(x[:, FEAT:], w)], axis=1).astype(BF16)


def _xw(X_r, gc1_W):
    return pl.pallas_call(
        _xw_body,
        grid=(STOCK // TM, NBT // 2),
        in_specs=[
            pl.BlockSpec((TM, 2 * FEAT), lambda i, j: (i, j)),
            pl.BlockSpec((FEAT, NHID), lambda i, j: (0, 0)),
        ],
        out_specs=pl.BlockSpec((TM, 2 * NHID), lambda i, j: (i, j)),
        out_shape=jax.ShapeDtypeStruct((STOCK, NBT * NHID), BF16),
    )(X_r, gc1_W)


def _u_body(adj_ref, hn_ref, xw_ref, b_ref, h1_ref, h2_ref):
    xw = xw_ref[...]
    b = b_ref[...]
    h1_ref[...] = jax.nn.relu(_dot(adj_ref[...], xw) + b).astype(BF16)
    h2_ref[...] = jax.nn.relu(_dot(hn_ref[...], xw) + b).astype(BF16)


def _u_stage(adj, H_new, XW_r, b1c):
    KW = NBT * NHID
    return pl.pallas_call(
        _u_body,
        grid=(STOCK // TM,),
        in_specs=[
            pl.BlockSpec((TM, STOCK), lambda i: (i, 0)),
            pl.BlockSpec((TM, STOCK), lambda i: (i, 0)),
            pl.BlockSpec((STOCK, KW), lambda i: (0, 0)),
            pl.BlockSpec((1, KW), lambda i: (0, 0)),
        ],
        out_specs=[
            pl.BlockSpec((TM, KW), lambda i: (i, 0)),
            pl.BlockSpec((TM, KW), lambda i: (i, 0)),
        ],
        out_shape=[
            jax.ShapeDtypeStruct((STOCK, KW), BF16),
            jax.ShapeDtypeStruct((STOCK, KW), BF16),
        ],
    )(adj, H_new, XW_r, b1c)


def _v_body(adj_ref, hn_ref, h1_ref, h2_ref, v_ref):
    v_ref[...] = _dot(adj_ref[...], h1_ref[...]) + _dot(hn_ref[...], h2_ref[...])


def _v_stage(adj, H_new, Ha, Hh):
    KW = NBT * NHID
    return pl.pallas_call(
        _v_body,
        grid=(STOCK // TM,),
        in_specs=[
            pl.BlockSpec((TM, STOCK), lambda i: (i, 0)),
            pl.BlockSpec((TM, STOCK), lambda i: (i, 0)),
            pl.BlockSpec((STOCK, KW), lambda i: (0, 0)),
            pl.BlockSpec((STOCK, KW), lambda i: (0, 0)),
        ],
        out_specs=pl.BlockSpec((TM, KW), lambda i: (i, 0)),
        out_shape=jax.ShapeDtypeStruct((STOCK, KW), F32),
    )(adj, H_new, Ha, Hh)


TMG = 512


def _gru_body(v_ref, wf0_ref, bf0_ref, whh0_ref, bhh0_ref,
              wih1_ref, bih1_ref, whh1_ref, bhh1_ref, out_ref):
    V = v_ref[...]
    wf0 = wf0_ref[...]
    bf0 = bf0_ref[...]
    whh0 = whh0_ref[...]
    bhh0 = bhh0_ref[...]
    wih1 = wih1_ref[...]
    bih1 = bih1_ref[...]
    whh1 = whh1_ref[...]
    bhh1 = bhh1_ref[...]

    def cell(gi, h, whh, bhh):
        gh = _dot(h, whh) + bhh
        r = jax.nn.sigmoid(gi[:, :RNN] + gh[:, :RNN])
        u = jax.nn.sigmoid(gi[:, RNN:2 * RNN] + gh[:, RNN:2 * RNN])
        c = jnp.tanh(gi[:, 2 * RNN:] + r * gh[:, 2 * RNN:])
        return (1.0 - u) * c + u * h

    h = jnp.zeros((TMG, RNN), F32)
    outs0 = []
    for t in range(T):
        gi = _dot(V[:, t * RNN:(t + 1) * RNN], wf0) + bf0
        h = cell(gi, h, whh0, bhh0)
        outs0.append(h)
    h = jnp.zeros((TMG, RNN), F32)
    for t in range(T):
        gi = _dot(outs0[t], wih1) + bih1
        h = cell(gi, h, whh1, bhh1)
        out_ref[:, t * RNN:(t + 1) * RNN] = h


def _gru(V, Wf0, bf0, Whh0T, bhh0, Wih1T, bih1, Whh1T, bhh1):
    TW = T * RNN
    wspec = pl.BlockSpec((RNN, 3 * RNN), lambda i, b: (0, 0))
    bspec = pl.BlockSpec((1, 3 * RNN), lambda i, b: (0, 0))
    return pl.pallas_call(
        _gru_body,
        grid=(STOCK // TMG, B),
        in_specs=[
            pl.BlockSpec((TMG, TW), lambda i, b: (i, b)),
            wspec, bspec, wspec, bspec, wspec, bspec, wspec, bspec,
        ],
        out_specs=pl.BlockSpec((TMG, TW),
                               lambda i, b: (b * (STOCK // TMG) + i, 0)),
        out_shape=jax.ShapeDtypeStruct((NROW, TW), F32),
    )(V, Wf0, bf0, Whh0T, bhh0, Wih1T, bih1, Whh1T, bhh1)


TMA = 256


def _attn_body(x_ref, wq_ref, wk_ref, wv_ref, fca_ref, hb_ref, o_ref):
    X = x_ref[...]
    wq = wq_ref[...]
    wk = wk_ref[...]
    wv = wv_ref[...]
    fca = fca_ref[...]
    HB = hb_ref[...]

    qs, ks, vs = [], [], []
    for t in range(T):
        rt = X[:, t * RNN:(t + 1) * RNN]
        qs.append(_dot(rt, wq))
        ks.append(_dot(rt, wk))
        vs.append(_dot(rt, wv))

    o_cols = []
    for i in range(T):
        srow = [_dot(qs[i] * ks[j], HB) for j in range(i + 1)]
        m = srow[0]
        for sj in srow[1:]:
            m = jnp.maximum(m, sj)
        es = [jnp.exp(sj - m) for sj in srow]
        tot = es[0]
        for ej in es[1:]:
            tot = tot + ej
        inv = 1.0 / tot
        o_i = es[0] * inv * vs[0]
        for j in range(1, i + 1):
            o_i = o_i + es[j] * inv * vs[j]
        o_cols.append(_dot(o_i, fca))
    o_ref[...] = jnp.concatenate(o_cols, axis=1)


def _attn(rnn_out, wq, wk, wv, fc_attn, HB):
    TW = T * RNN
    wspec = pl.BlockSpec((RNN, RNN), lambda i: (0, 0))
    return pl.pallas_call(
        _attn_body,
        grid=(NROW // TMA,),
        in_specs=[
            pl.BlockSpec((TMA, TW), lambda i: (i, 0)),
            wspec, wspec, wspec, wspec, wspec,
        ],
        out_specs=pl.BlockSpec((TMA, TW), lambda i: (i, 0)),
        out_shape=jax.ShapeDtypeStruct((NROW, TW), F32),
    )(rnn_out, wq, wk, wv, fc_attn, HB)


def _mhgn_body(h_ref, ht_ref, adj_ref, o_ref):
    o_ref[...] = adj_ref[...] + _dot(h_ref[...], ht_ref[...]) * (1.0 / NEDGE)


def _mhgn(H, H_T, adj):
    return pl.pallas_call(
        _mhgn_body,
        grid=(STOCK // TM,),
        in_specs=[
            pl.BlockSpec((TM, NEDGE), lambda i: (i, 0)),
            pl.BlockSpec((NEDGE, STOCK), lambda i: (0, 0)),
            pl.BlockSpec((TM, STOCK), lambda i: (i, 0)),
        ],
        out_specs=pl.BlockSpec((TM, STOCK), lambda i: (i, 0)),
        out_shape=jax.ShapeDtypeStruct((STOCK, STOCK), F32),
    )(H, H_T, adj)


def _uenc_body(oc0_ref, oc1_ref, r0_ref, r1_ref, lng_ref, lnb_ref,
               w_ref, o_ref):
    w = w_ref[...]
    g = lng_ref[...]
    bb = lnb_ref[...]

    def enc_block(oc, rn):
        cols = []
        for t in range(T):
            et = oc[:, t * RNN:(t + 1) * RNN] + rn[:, t * RNN:(t + 1) * RNN]
            mu = jnp.mean(et, axis=-1, keepdims=True)
            var = jnp.mean((et - mu) ** 2, axis=-1, keepdims=True)
            cols.append(g * (et - mu) / jnp.sqrt(var + 1e-6) + bb)
        return _dot(jnp.concatenate(cols, axis=1), w)

    o_ref[...] = jnp.concatenate(
        [enc_block(oc0_ref[...], r0_ref[...]),
         enc_block(oc1_ref[...], r1_ref[...])], axis=1)


def _uenc(oc, rnn_out, ln_g2, ln_b2, Wh1):
    TW = T * RNN
    nb = STOCK // TM
    lo = pl.BlockSpec((TM, TW), lambda i: (i, 0))
    hi = pl.BlockSpec((TM, TW), lambda i: (nb + i, 0))
    vspec = pl.BlockSpec((1, RNN), lambda i: (0, 0))
    return pl.pallas_call(
        _uenc_body,
        grid=(nb,),
        in_specs=[
            lo, hi, lo, hi, vspec, vspec,
            pl.BlockSpec((TW, NHID), lambda i: (0, 0)),
        ],
        out_specs=pl.BlockSpec((TM, B * NHID), lambda i: (i, 0)),
        out_shape=jax.ShapeDtypeStruct((STOCK, B * NHID), F32),
    )(oc, oc, rnn_out, rnn_out, ln_g2, ln_b2, Wh1)


def _hgn_body(m_ref, u_ref, b_ref, o_ref):
    o_ref[...] = jax.nn.relu(_dot(m_ref[...], u_ref[...]) + b_ref[...])


def _hgn(M, U, bh1c):
    KW = B * NHID
    return pl.pallas_call(
        _hgn_body,
        grid=(STOCK // TM,),
        in_specs=[
            pl.BlockSpec((TM, STOCK), lambda i: (i, 0)),
            pl.BlockSpec((STOCK, KW), lambda i: (0, 0)),
            pl.BlockSpec((1, KW), lambda i: (0, 0)),
        ],
        out_specs=pl.BlockSpec((TM, KW), lambda i: (i, 0)),
        out_shape=jax.ShapeDtypeStruct((STOCK, KW), F32),
    )(M, U, bh1c)


def _final_body(hg_ref, avw_ref, avb_ref, avu_ref, lw_ref, lb_ref,
                fcwh_ref, fcwa_ref, fcb_ref, wprj_ref, seq_ref, pred_ref):
    Hg = hg_ref[...]
    a_laten = jnp.tanh(_dot(Hg, avw_ref[...]) + avb_ref[...])
    s = jnp.sum(a_laten * avu_ref[...], axis=1, keepdims=True)
    m = jnp.max(s)
    e = jnp.exp(s - m)
    alph = e / jnp.sum(e)
    acs = alph * jnp.sum(Hg, axis=1, keepdims=True)
    a_con = acs * lw_ref[...] + lb_ref[...]
    pred_ref[...] = _dot(Hg, fcwh_ref[...]) + _dot(a_con, fcwa_ref[...]) + fcb_ref[...]
    seq_ref[...] = _dot(Hg, wprj_ref[...]) * (RNN ** -0.5)


def _final(hgn_flat, av_w, av_b2, av_u2, L_W, L_b2, fcW_h, fcW_a, fc_b2, W_prjT):
    return pl.pallas_call(
        _final_body,
        out_shape=[
            jax.ShapeDtypeStruct((NROW, NCLASS), F32),
            jax.ShapeDtypeStruct((NROW, NCLASS), F32),
        ],
    )(hgn_flat, av_w, av_b2, av_u2, L_W, L_b2, fcW_h, fcW_a, fc_b2, W_prjT)


def kernel(src_seq, H, adj, n_hid, gc1_W, gc1_b, gc2_W, gc2_b, m_W, m_b,
           lin_W, lin_b, gru_Wih0, gru_Whh0, gru_bih0, gru_bhh0,
           gru_Wih1, gru_Whh1, gru_bih1, gru_bhh1, wq, wk, wv, fc_attn,
           ln_g, ln_b, Wh1, bh1, av_w, av_b, av_u, L_W, L_b, fc_W, fc_b,
           W_prj):
    del n_hid

    W2L = gc2_W @ lin_W
    bias2L = 2.0 * (gc2_b @ lin_W) + lin_b
    Wf0 = W2L @ gru_Wih0.T
    bf0 = (bias2L @ gru_Wih0.T + gru_bih0)[None, :]
    Whh0T = gru_Whh0.T
    Wih1T = gru_Wih1.T
    Whh1T = gru_Whh1.T
    bhh0 = gru_bhh0[None, :]
    bih1 = gru_bih1[None, :]
    bhh1 = gru_bhh1[None, :]

    X_r = jnp.transpose(src_seq, (1, 0, 2, 3)).reshape(STOCK, NBT * FEAT)
    b1c = jnp.tile(gc1_b, (NBT,))[None, :]
    adj_bf = adj.astype(BF16)
    H_new = _hnew(H, m_W, m_b[None, :])
    XW_r = _xw(X_r, gc1_W)
    Ha, Hh = _u_stage(adj_bf, H_new, XW_r, b1c)
    V = _v_stage(adj_bf, H_new, Ha, Hh)

    rnn_out = _gru(V, Wf0, bf0, Whh0T, bhh0, Wih1T, bih1, Whh1T, bhh1)
    hb = jnp.repeat(jnp.repeat(jnp.eye(NHEAD, dtype=F32), DK, axis=0),
                    DK, axis=1) * (1.0 / np.sqrt(DK))
    oc = _attn(rnn_out, wq, wk, wv, fc_attn, hb)

    M = _mhgn(H, H.T, adj)
    U = _uenc(oc, rnn_out, ln_g[None, :], ln_b[None, :], Wh1)
    hgn_cols = _hgn(M, U, jnp.tile(bh1, (B,))[None, :])
    hgn_flat = jnp.transpose(
        hgn_cols.reshape(STOCK, B, NHID), (1, 0, 2)).reshape(NROW, NHID)

    seq_logit, pred = _final(
        hgn_flat, av_w, av_b[None, :], av_u[None, :], L_W, L_b[None, :],
        fc_W[:NHID, :], fc_W[NHID:, :], fc_b[None, :], W_prj.T)
    return (seq_logit, pred)


# speedup vs baseline: 5.1581x; 1.5943x over previous
import jax
import jax.numpy as jnp
import numpy as np
from jax.experimental import pallas as pl
from jax.experimental.pallas import tpu as pltpu

B, STOCK, T, FEAT = 2, 2048, 8, 128
NHID, RNN, NHEAD, DK, DV = 64, 64, 4, 16, 16
NEDGE = 256
NCLASS = 3
NBT = B * T
NROW = B * STOCK
TM = 256
TMG = 512

F32 = jnp.float32
BF16 = jnp.bfloat16


def _dot(a, b):
    return jnp.dot(a, b, preferred_element_type=F32)


def _xw_body(x_ref, w_ref, o_ref):
    x = x_ref[...].reshape(TM, T * FEAT)
    w = w_ref[...]
    cols = [_dot(x[:, t * FEAT:(t + 1) * FEAT], w) for t in range(T)]
    o_ref[...] = jnp.concatenate(cols, axis=1).astype(BF16)


def _xw(src_seq, gc1_W):
    return pl.pallas_call(
        _xw_body,
        grid=(STOCK // TM, B),
        in_specs=[
            pl.BlockSpec((1, TM, T, FEAT), lambda i, b: (b, i, 0, 0)),
            pl.BlockSpec((FEAT, NHID), lambda i, b: (0, 0)),
        ],
        out_specs=pl.BlockSpec((TM, T * NHID), lambda i, b: (i, b)),
        out_shape=jax.ShapeDtypeStruct((STOCK, NBT * NHID), BF16),
    )(src_seq, gc1_W)


def _u_body(h_ref, mw_ref, mb_ref, adj_ref, xw_ref, b_ref, h1_ref, h2_ref):
    hn = (_dot(h_ref[...], mw_ref[...]) + mb_ref[...]).astype(BF16)
    xw = xw_ref[...]
    b = b_ref[...]
    h1_ref[...] = jax.nn.relu(_dot(adj_ref[...], xw) + b).astype(BF16)
    h2_ref[...] = jax.nn.relu(_dot(hn, xw) + b).astype(BF16)


def _u_stage(H, m_W, m_b2, adj, XW_r, b1c):
    KW = NBT * NHID
    return pl.pallas_call(
        _u_body,
        grid=(STOCK // TM,),
        in_specs=[
            pl.BlockSpec((TM, NEDGE), lambda i: (i, 0)),
            pl.BlockSpec((NEDGE, STOCK), lambda i: (0, 0)),
            pl.BlockSpec((1, STOCK), lambda i: (0, 0)),
            pl.BlockSpec((TM, STOCK), lambda i: (i, 0)),
            pl.BlockSpec((STOCK, KW), lambda i: (0, 0)),
            pl.BlockSpec((1, KW), lambda i: (0, 0)),
        ],
        out_specs=[
            pl.BlockSpec((TM, KW), lambda i: (i, 0)),
            pl.BlockSpec((TM, KW), lambda i: (i, 0)),
        ],
        out_shape=[
            jax.ShapeDtypeStruct((STOCK, KW), BF16),
            jax.ShapeDtypeStruct((STOCK, KW), BF16),
        ],
    )(H, m_W, m_b2, adj, XW_r, b1c)


def _v_body(h_ref, mw_ref, mb_ref, adj_ref, h1_ref, h2_ref, v_ref):
    hn = (_dot(h_ref[...], mw_ref[...]) + mb_ref[...]).astype(BF16)
    v_ref[...] = _dot(adj_ref[...], h1_ref[...]) + _dot(hn, h2_ref[...])


def _v_stage(H, m_W, m_b2, adj, Ha, Hh):
    KW = NBT * NHID
    return pl.pallas_call(
        _v_body,
        grid=(STOCK // TM,),
        in_specs=[
            pl.BlockSpec((TM, NEDGE), lambda i: (i, 0)),
            pl.BlockSpec((NEDGE, STOCK), lambda i: (0, 0)),
            pl.BlockSpec((1, STOCK), lambda i: (0, 0)),
            pl.BlockSpec((TM, STOCK), lambda i: (i, 0)),
            pl.BlockSpec((STOCK, KW), lambda i: (0, 0)),
            pl.BlockSpec((STOCK, KW), lambda i: (0, 0)),
        ],
        out_specs=pl.BlockSpec((TM, KW), lambda i: (i, 0)),
        out_shape=jax.ShapeDtypeStruct((STOCK, KW), F32),
    )(H, m_W, m_b2, adj, Ha, Hh)


def _seq_body(v_ref, wf0_ref, bf0_ref, whh0_ref, bhh0_ref,
              wih1_ref, bih1_ref, whh1_ref, bhh1_ref,
              wq_ref, wk_ref, wv_ref, fca_ref, hb_ref,
              lng_ref, lnb_ref, wh1_ref, o_ref):
    V = v_ref[...]
    wf0 = wf0_ref[...]
    bf0 = bf0_ref[...]
    whh0 = whh0_ref[...]
    bhh0 = bhh0_ref[...]
    wih1 = wih1_ref[...]
    bih1 = bih1_ref[...]
    whh1 = whh1_ref[...]
    bhh1 = bhh1_ref[...]

    def cell(gi, h, whh, bhh):
        gh = _dot(h, whh) + bhh
        r = jax.nn.sigmoid(gi[:, :RNN] + gh[:, :RNN])
        u = jax.nn.sigmoid(gi[:, RNN:2 * RNN] + gh[:, RNN:2 * RNN])
        c = jnp.tanh(gi[:, 2 * RNN:] + r * gh[:, 2 * RNN:])
        return (1.0 - u) * c + u * h

    h = jnp.zeros((TMG, RNN), F32)
    outs0 = []
    for t in range(T):
        gi = _dot(V[:, t * RNN:(t + 1) * RNN], wf0) + bf0
        h = cell(gi, h, whh0, bhh0)
        outs0.append(h)
    h = jnp.zeros((TMG, RNN), F32)
    rnn = []
    for t in range(T):
        gi = _dot(outs0[t], wih1) + bih1
        h = cell(gi, h, whh1, bhh1)
        rnn.append(h)

    wq = wq_ref[...]
    wk = wk_ref[...]
    wv = wv_ref[...]
    fca = fca_ref[...]
    HB = hb_ref[...]
    qs = [_dot(rt, wq) for rt in rnn]
    ks = [_dot(rt, wk) for rt in rnn]
    vs = [_dot(rt, wv) for rt in rnn]
    g = lng_ref[...]
    bb = lnb_ref[...]
    enc_cols = []
    for i in range(T):
        srow = [_dot(qs[i] * ks[j], HB) for j in range(i + 1)]
        m = srow[0]
        for sj in srow[1:]:
            m = jnp.maximum(m, sj)
        es = [jnp.exp(sj - m) for sj in srow]
        tot = es[0]
        for ej in es[1:]:
            tot = tot + ej
        inv = 1.0 / tot
        o_i = es[0] * inv * vs[0]
        for j in range(1, i + 1):
            o_i = o_i + es[j] * inv * vs[j]
        et = _dot(o_i, fca) + rnn[i]
        mu = jnp.mean(et, axis=-1, keepdims=True)
        var = jnp.mean((et - mu) ** 2, axis=-1, keepdims=True)
        enc_cols.append(g * (et - mu) / jnp.sqrt(var + 1e-6) + bb)

    enc = jnp.concatenate(enc_cols, axis=1)
    o_ref[...] = _dot(enc, wh1_ref[...])[None]


def _seq_stage(V, Wf0, bf0, Whh0T, bhh0, Wih1T, bih1, Whh1T, bhh1,
               wq, wk, wv, fc_attn, HB, ln_g2, ln_b2, Wh1):
    TW = T * RNN
    wspec = pl.BlockSpec((RNN, 3 * RNN), lambda i, b: (0, 0))
    bspec = pl.BlockSpec((1, 3 * RNN), lambda i, b: (0, 0))
    sspec = pl.BlockSpec((RNN, RNN), lambda i, b: (0, 0))
    vspec = pl.BlockSpec((1, RNN), lambda i, b: (0, 0))
    return pl.pallas_call(
        _seq_body,
        grid=(STOCK // TMG, B),
        in_specs=[
            pl.BlockSpec((TMG, TW), lambda i, b: (i, b)),
            wspec, bspec, wspec, bspec, wspec, bspec, wspec, bspec,
            sspec, sspec, sspec, sspec, sspec, vspec, vspec,
            pl.BlockSpec((TW, NHID), lambda i, b: (0, 0)),
        ],
        out_specs=pl.BlockSpec((1, TMG, NHID), lambda i, b: (b, i, 0)),
        out_shape=jax.ShapeDtypeStruct((B, STOCK, NHID), F32),
    )(V, Wf0, bf0, Whh0T, bhh0, Wih1T, bih1, Whh1T, bhh1,
      wq, wk, wv, fc_attn, HB, ln_g2, ln_b2, Wh1)


def _hgn_body(h_ref, ht_ref, adj_ref, u0_ref, u1_ref, b_ref, o_ref):
    M = adj_ref[...] + _dot(h_ref[...], ht_ref[...]) * (1.0 / NEDGE)
    u = jnp.concatenate([u0_ref[0], u1_ref[0]], axis=1)
    o_ref[...] = jax.nn.relu(_dot(M, u) + b_ref[...])


def _hgn(H, H_T, adj, U3, bh1c):
    KW = B * NHID
    return pl.pallas_call(
        _hgn_body,
        grid=(STOCK // TM,),
        in_specs=[
            pl.BlockSpec((TM, NEDGE), lambda i: (i, 0)),
            pl.BlockSpec((NEDGE, STOCK), lambda i: (0, 0)),
            pl.BlockSpec((TM, STOCK), lambda i: (i, 0)),
            pl.BlockSpec((1, STOCK, NHID), lambda i: (0, 0, 0)),
            pl.BlockSpec((1, STOCK, NHID), lambda i: (1, 0, 0)),
            pl.BlockSpec((1, KW), lambda i: (0, 0)),
        ],
        out_specs=pl.BlockSpec((TM, KW), lambda i: (i, 0)),
        out_shape=jax.ShapeDtypeStruct((STOCK, KW), F32),
    )(H, H_T, adj, U3, U3, bh1c)


def _final_body(hg_ref, avw_ref, avb_ref, avu_ref, lw_ref, lb_ref,
                fcwh_ref, fcwa_ref, fcb_ref, wprj_ref, seq_ref, pred_ref):
    Hg = hg_ref[...]
    a_laten = jnp.tanh(_dot(Hg, avw_ref[...]) + avb_ref[...])
    s = jnp.sum(a_laten * avu_ref[...], axis=1, keepdims=True)
    m = jnp.max(s)
    e = jnp.exp(s - m)
    alph = e / jnp.sum(e)
    acs = alph * jnp.sum(Hg, axis=1, keepdims=True)
    a_con = acs * lw_ref[...] + lb_ref[...]
    pred_ref[...] = _dot(Hg, fcwh_ref[...]) + _dot(a_con, fcwa_ref[...]) + fcb_ref[...]
    seq_ref[...] = _dot(Hg, wprj_ref[...]) * (RNN ** -0.5)


def _final(hgn_flat, av_w, av_b2, av_u2, L_W, L_b2, fcW_h, fcW_a, fc_b2, W_prjT):
    return pl.pallas_call(
        _final_body,
        out_shape=[
            jax.ShapeDtypeStruct((NROW, NCLASS), F32),
            jax.ShapeDtypeStruct((NROW, NCLASS), F32),
        ],
    )(hgn_flat, av_w, av_b2, av_u2, L_W, L_b2, fcW_h, fcW_a, fc_b2, W_prjT)


def kernel(src_seq, H, adj, n_hid, gc1_W, gc1_b, gc2_W, gc2_b, m_W, m_b,
           lin_W, lin_b, gru_Wih0, gru_Whh0, gru_bih0, gru_bhh0,
           gru_Wih1, gru_Whh1, gru_bih1, gru_bhh1, wq, wk, wv, fc_attn,
           ln_g, ln_b, Wh1, bh1, av_w, av_b, av_u, L_W, L_b, fc_W, fc_b,
           W_prj):
    del n_hid

    W2L = gc2_W @ lin_W
    bias2L = 2.0 * (gc2_b @ lin_W) + lin_b
    Wf0 = W2L @ gru_Wih0.T
    bf0 = (bias2L @ gru_Wih0.T + gru_bih0)[None, :]
    hb = jnp.repeat(jnp.repeat(jnp.eye(NHEAD, dtype=F32), DK, axis=0),
                    DK, axis=1) * (1.0 / np.sqrt(DK))

    b1c = jnp.tile(gc1_b, (NBT,))[None, :]
    adj_bf = adj.astype(BF16)
    m_b2 = m_b[None, :]

    XW_r = _xw(src_seq, gc1_W)
    Ha, Hh = _u_stage(H, m_W, m_b2, adj_bf, XW_r, b1c)
    V = _v_stage(H, m_W, m_b2, adj_bf, Ha, Hh)

    U3 = _seq_stage(V, Wf0, bf0, gru_Whh0.T, gru_bhh0[None, :],
                    gru_Wih1.T, gru_bih1[None, :], gru_Whh1.T,
                    gru_bhh1[None, :], wq, wk, wv, fc_attn, hb,
                    ln_g[None, :], ln_b[None, :], Wh1)

    hgn_cols = _hgn(H, H.T, adj, U3, jnp.tile(bh1, (B,))[None, :])
    hgn_flat = jnp.transpose(
        hgn_cols.reshape(STOCK, B, NHID), (1, 0, 2)).reshape(NROW, NHID)

    seq_logit, pred = _final(
        hgn_flat, av_w, av_b[None, :], av_u[None, :], L_W, L_b[None, :],
        fc_W[:NHID, :], fc_W[NHID:, :], fc_b[None, :], W_prj.T)
    return (seq_logit, pred)


# speedup vs baseline: 5.4027x; 1.0474x over previous
import jax
import jax.numpy as jnp
import numpy as np
from jax.experimental import pallas as pl
from jax.experimental.pallas import tpu as pltpu

B, STOCK, T, FEAT = 2, 2048, 8, 128
NHID, RNN, NHEAD, DK, DV = 64, 64, 4, 16, 16
NEDGE = 256
NCLASS = 3
NBT = B * T
NROW = B * STOCK
TM = 512
TMG = 512

F32 = jnp.float32
BF16 = jnp.bfloat16


def _dot(a, b):
    return jnp.dot(a, b, preferred_element_type=F32)


def _xw_body(x_ref, w_ref, o_ref):
    x = x_ref[...].reshape(TM, T * FEAT)
    w = w_ref[...]
    cols = [_dot(x[:, t * FEAT:(t + 1) * FEAT], w) for t in range(T)]
    o_ref[...] = jnp.concatenate(cols, axis=1).astype(BF16)


def _xw(src_seq, gc1_W):
    return pl.pallas_call(
        _xw_body,
        grid=(STOCK // TM, B),
        in_specs=[
            pl.BlockSpec((1, TM, T, FEAT), lambda i, b: (b, i, 0, 0)),
            pl.BlockSpec((FEAT, NHID), lambda i, b: (0, 0)),
        ],
        out_specs=pl.BlockSpec((TM, T * NHID), lambda i, b: (i, b)),
        out_shape=jax.ShapeDtypeStruct((STOCK, NBT * NHID), BF16),
    )(src_seq, gc1_W)


def _u_body(h_ref, mw_ref, mb_ref, adj_ref, xw_ref, b_ref, h1_ref, h2_ref):
    hn = (_dot(h_ref[...], mw_ref[...]) + mb_ref[...]).astype(BF16)
    xw = xw_ref[...]
    b = b_ref[...]
    h1_ref[...] = jax.nn.relu(_dot(adj_ref[...], xw) + b).astype(BF16)
    h2_ref[...] = jax.nn.relu(_dot(hn, xw) + b).astype(BF16)


def _u_stage(H, m_W, m_b2, adj, XW_r, b1c):
    KW = NBT * NHID
    return pl.pallas_call(
        _u_body,
        grid=(STOCK // TM,),
        in_specs=[
            pl.BlockSpec((TM, NEDGE), lambda i: (i, 0)),
            pl.BlockSpec((NEDGE, STOCK), lambda i: (0, 0)),
            pl.BlockSpec((1, STOCK), lambda i: (0, 0)),
            pl.BlockSpec((TM, STOCK), lambda i: (i, 0)),
            pl.BlockSpec((STOCK, KW), lambda i: (0, 0)),
            pl.BlockSpec((1, KW), lambda i: (0, 0)),
        ],
        out_specs=[
            pl.BlockSpec((TM, KW), lambda i: (i, 0)),
            pl.BlockSpec((TM, KW), lambda i: (i, 0)),
        ],
        out_shape=[
            jax.ShapeDtypeStruct((STOCK, KW), BF16),
            jax.ShapeDtypeStruct((STOCK, KW), BF16),
        ],
    )(H, m_W, m_b2, adj, XW_r, b1c)


def _v_body(h_ref, mw_ref, mb_ref, adj_ref, h1_ref, h2_ref, v_ref):
    hn = (_dot(h_ref[...], mw_ref[...]) + mb_ref[...]).astype(BF16)
    v_ref[...] = _dot(adj_ref[...], h1_ref[...]) + _dot(hn, h2_ref[...])


def _v_stage(H, m_W, m_b2, adj, Ha, Hh):
    KW = NBT * NHID
    return pl.pallas_call(
        _v_body,
        grid=(STOCK // TM,),
        in_specs=[
            pl.BlockSpec((TM, NEDGE), lambda i: (i, 0)),
            pl.BlockSpec((NEDGE, STOCK), lambda i: (0, 0)),
            pl.BlockSpec((1, STOCK), lambda i: (0, 0)),
            pl.BlockSpec((TM, STOCK), lambda i: (i, 0)),
            pl.BlockSpec((STOCK, KW), lambda i: (0, 0)),
            pl.BlockSpec((STOCK, KW), lambda i: (0, 0)),
        ],
        out_specs=pl.BlockSpec((TM, KW), lambda i: (i, 0)),
        out_shape=jax.ShapeDtypeStruct((STOCK, KW), F32),
    )(H, m_W, m_b2, adj, Ha, Hh)


def _seq_body(v_ref, wf0_ref, bf0_ref, whh0_ref, bhh0_ref,
              wih1_ref, bih1_ref, whh1_ref, bhh1_ref,
              wq_ref, wk_ref, wv_ref, fca_ref, hb_ref,
              lng_ref, lnb_ref, wh1_ref, o_ref):
    V = v_ref[...]
    wf0 = wf0_ref[...]
    bf0 = bf0_ref[...]
    whh0 = whh0_ref[...]
    bhh0 = bhh0_ref[...]
    wih1 = wih1_ref[...]
    bih1 = bih1_ref[...]
    whh1 = whh1_ref[...]
    bhh1 = bhh1_ref[...]

    def cell(gi, h, whh, bhh):
        gh = _dot(h, whh) + bhh
        r = jax.nn.sigmoid(gi[:, :RNN] + gh[:, :RNN])
        u = jax.nn.sigmoid(gi[:, RNN:2 * RNN] + gh[:, RNN:2 * RNN])
        c = jnp.tanh(gi[:, 2 * RNN:] + r * gh[:, 2 * RNN:])
        return (1.0 - u) * c + u * h

    h = jnp.zeros((TMG, RNN), F32)
    outs0 = []
    for t in range(T):
        gi = _dot(V[:, t * RNN:(t + 1) * RNN], wf0) + bf0
        h = cell(gi, h, whh0, bhh0)
        outs0.append(h)
    h = jnp.zeros((TMG, RNN), F32)
    rnn = []
    for t in range(T):
        gi = _dot(outs0[t], wih1) + bih1
        h = cell(gi, h, whh1, bhh1)
        rnn.append(h)

    wq = wq_ref[...]
    wk = wk_ref[...]
    wv = wv_ref[...]
    fca = fca_ref[...]
    HB = hb_ref[...]
    qs = [_dot(rt, wq) for rt in rnn]
    ks = [_dot(rt, wk) for rt in rnn]
    vs = [_dot(rt, wv) for rt in rnn]
    g = lng_ref[...]
    bb = lnb_ref[...]
    enc_cols = []
    for i in range(T):
        srow = [_dot(qs[i] * ks[j], HB) for j in range(i + 1)]
        m = srow[0]
        for sj in srow[1:]:
            m = jnp.maximum(m, sj)
        es = [jnp.exp(sj - m) for sj in srow]
        tot = es[0]
        for ej in es[1:]:
            tot = tot + ej
        inv = 1.0 / tot
        o_i = es[0] * inv * vs[0]
        for j in range(1, i + 1):
            o_i = o_i + es[j] * inv * vs[j]
        et = _dot(o_i, fca) + rnn[i]
        mu = jnp.mean(et, axis=-1, keepdims=True)
        var = jnp.mean((et - mu) ** 2, axis=-1, keepdims=True)
        enc_cols.append(g * (et - mu) / jnp.sqrt(var + 1e-6) + bb)

    enc = jnp.concatenate(enc_cols, axis=1)
    o_ref[...] = _dot(enc, wh1_ref[...])[None]


def _seq_stage(V, Wf0, bf0, Whh0T, bhh0, Wih1T, bih1, Whh1T, bhh1,
               wq, wk, wv, fc_attn, HB, ln_g2, ln_b2, Wh1):
    TW = T * RNN
    wspec = pl.BlockSpec((RNN, 3 * RNN), lambda i, b: (0, 0))
    bspec = pl.BlockSpec((1, 3 * RNN), lambda i, b: (0, 0))
    sspec = pl.BlockSpec((RNN, RNN), lambda i, b: (0, 0))
    vspec = pl.BlockSpec((1, RNN), lambda i, b: (0, 0))
    return pl.pallas_call(
        _seq_body,
        grid=(STOCK // TMG, B),
        in_specs=[
            pl.BlockSpec((TMG, TW), lambda i, b: (i, b)),
            wspec, bspec, wspec, bspec, wspec, bspec, wspec, bspec,
            sspec, sspec, sspec, sspec, sspec, vspec, vspec,
            pl.BlockSpec((TW, NHID), lambda i, b: (0, 0)),
        ],
        out_specs=pl.BlockSpec((1, TMG, NHID), lambda i, b: (b, i, 0)),
        out_shape=jax.ShapeDtypeStruct((B, STOCK, NHID), F32),
    )(V, Wf0, bf0, Whh0T, bhh0, Wih1T, bih1, Whh1T, bhh1,
      wq, wk, wv, fc_attn, HB, ln_g2, ln_b2, Wh1)


def _hgn_body(h_ref, ht_ref, adj_ref, u0_ref, u1_ref, b_ref, o_ref):
    M = (adj_ref[...].astype(F32)
         + _dot(h_ref[...], ht_ref[...]) * (1.0 / NEDGE)).astype(BF16)
    u = jnp.concatenate([u0_ref[0], u1_ref[0]], axis=1).astype(BF16)
    o_ref[...] = jax.nn.relu(_dot(M, u) + b_ref[...])


def _hgn(H, H_T, adj, U3, bh1c):
    KW = B * NHID
    return pl.pallas_call(
        _hgn_body,
        grid=(STOCK // TM,),
        in_specs=[
            pl.BlockSpec((TM, NEDGE), lambda i: (i, 0)),
            pl.BlockSpec((NEDGE, STOCK), lambda i: (0, 0)),
            pl.BlockSpec((TM, STOCK), lambda i: (i, 0)),
            pl.BlockSpec((1, STOCK, NHID), lambda i: (0, 0, 0)),
            pl.BlockSpec((1, STOCK, NHID), lambda i: (1, 0, 0)),
            pl.BlockSpec((1, KW), lambda i: (0, 0)),
        ],
        out_specs=pl.BlockSpec((TM, KW), lambda i: (i, 0)),
        out_shape=jax.ShapeDtypeStruct((STOCK, KW), F32),
    )(H, H_T, adj, U3, U3, bh1c)


def _final_body(hg_ref, avw_ref, avb_ref, avu_ref, lw_ref, lb_ref,
                fcwh_ref, fcwa_ref, fcb_ref, wprj_ref, seq_ref, pred_ref):
    Hg = hg_ref[...]
    a_laten = jnp.tanh(_dot(Hg, avw_ref[...]) + avb_ref[...])
    s = jnp.sum(a_laten * avu_ref[...], axis=1, keepdims=True)
    m = jnp.max(s)
    e = jnp.exp(s - m)
    alph = e / jnp.sum(e)
    acs = alph * jnp.sum(Hg, axis=1, keepdims=True)
    a_con = acs * lw_ref[...] + lb_ref[...]
    pred_ref[...] = _dot(Hg, fcwh_ref[...]) + _dot(a_con, fcwa_ref[...]) + fcb_ref[...]
    seq_ref[...] = _dot(Hg, wprj_ref[...]) * (RNN ** -0.5)


def _final(hgn_flat, av_w, av_b2, av_u2, L_W, L_b2, fcW_h, fcW_a, fc_b2, W_prjT):
    return pl.pallas_call(
        _final_body,
        out_shape=[
            jax.ShapeDtypeStruct((NROW, NCLASS), F32),
            jax.ShapeDtypeStruct((NROW, NCLASS), F32),
        ],
    )(hgn_flat, av_w, av_b2, av_u2, L_W, L_b2, fcW_h, fcW_a, fc_b2, W_prjT)


def kernel(src_seq, H, adj, n_hid, gc1_W, gc1_b, gc2_W, gc2_b, m_W, m_b,
           lin_W, lin_b, gru_Wih0, gru_Whh0, gru_bih0, gru_bhh0,
           gru_Wih1, gru_Whh1, gru_bih1, gru_bhh1, wq, wk, wv, fc_attn,
           ln_g, ln_b, Wh1, bh1, av_w, av_b, av_u, L_W, L_b, fc_W, fc_b,
           W_prj):
    del n_hid

    W2L = gc2_W @ lin_W
    bias2L = 2.0 * (gc2_b @ lin_W) + lin_b
    Wf0 = W2L @ gru_Wih0.T
    bf0 = (bias2L @ gru_Wih0.T + gru_bih0)[None, :]
    hb = jnp.repeat(jnp.repeat(jnp.eye(NHEAD, dtype=F32), DK, axis=0),
                    DK, axis=1) * (1.0 / np.sqrt(DK))

    b1c = jnp.tile(gc1_b, (NBT,))[None, :]
    adj_bf = adj.astype(BF16)
    m_b2 = m_b[None, :]

    XW_r = _xw(src_seq, gc1_W)
    Ha, Hh = _u_stage(H, m_W, m_b2, adj_bf, XW_r, b1c)
    V = _v_stage(H, m_W, m_b2, adj_bf, Ha, Hh)

    U3 = _seq_stage(V, Wf0, bf0, gru_Whh0.T, gru_bhh0[None, :],
                    gru_Wih1.T, gru_bih1[None, :], gru_Whh1.T,
                    gru_bhh1[None, :], wq, wk, wv, fc_attn, hb,
                    ln_g[None, :], ln_b[None, :], Wh1)

    hgn_cols = _hgn(H, H.T, adj_bf, U3, jnp.tile(bh1, (B,))[None, :])
    hgn_flat = jnp.transpose(
        hgn_cols.reshape(STOCK, B, NHID), (1, 0, 2)).reshape(NROW, NHID)

    seq_logit, pred = _final(
        hgn_flat, av_w, av_b[None, :], av_u[None, :], L_W, L_b[None, :],
        fc_W[:NHID, :], fc_W[NHID:, :], fc_b[None, :], W_prj.T)
    return (seq_logit, pred)
